# baseline (speedup 1.0000x reference)
import functools

import jax
import jax.numpy as jnp
from jax import lax
from jax.experimental import pallas as pl
from jax.experimental.pallas import tpu as pltpu
from jax.experimental.pallas import tpu_sc as plsc

_LANES = 16


def _sc_body(b, h, w, d, tab_hbm, out_hbm, tsp, buf, sems):
    c = lax.axis_index("c")
    s = lax.axis_index("s")
    wid = c * 16 + s
    ofs = s * _LANES
    pltpu.sync_copy(tab_hbm, tsp)
    hw = h * w
    chunks = hw // _LANES

    def build_row_x(j, carry):
        base = (ofs + j) * w
        va = tsp[pl.ds(base, _LANES)]
        vb = tsp[pl.ds(base + _LANES, _LANES)]
        for k in range(chunks):
            src = va if (k % (w // _LANES)) == 0 else vb
            buf[pl.ds(j * hw + k * _LANES, _LANES)] = src
        return carry

    def build_row_y(j, carry):
        base = d * w + (ofs + j) * h
        va = tsp[pl.ds(base, _LANES)]
        vb = tsp[pl.ds(base + _LANES, _LANES)]
        for k in range(chunks):
            hh = k // (w // _LANES)
            src = va if hh < _LANES else vb
            idx = jnp.zeros((_LANES,), jnp.int32) + (hh % _LANES)
            val = lax.gather(
                src, idx[:, None],
                lax.GatherDimensionNumbers(
                    offset_dims=(), collapsed_slice_dims=(0,),
                    start_index_map=(0,)),
                slice_sizes=(1,),
                mode=lax.GatherScatterMode.PROMISE_IN_BOUNDS)
            buf[pl.ds(j * hw + k * _LANES, _LANES)] = val
        return carry

    @pl.when(c == 0)
    def _x():
        lax.fori_loop(0, _LANES, build_row_x, 0)

    @pl.when(c == 1)
    def _y():
        lax.fori_loop(0, _LANES, build_row_y, 0)

    blk = _LANES * hw
    start0 = wid * blk
    handles = [
        pltpu.async_copy(
            buf, out_hbm.at[pl.ds(i * (2 * d * hw) + start0, blk)], sems)
        for i in range(b)
    ]
    for hnd in handles:
        hnd.wait()


def kernel(pixel_values, row_embedding, column_embedding):
    b = pixel_values.shape[0]
    h, w = pixel_values.shape[-2], pixel_values.shape[-1]
    d = row_embedding.shape[-1]
    tab = jnp.stack([column_embedding[:w].T, row_embedding[:h].T]).reshape(-1)
    mesh = plsc.VectorSubcoreMesh(core_axis_name="c", subcore_axis_name="s")
    body = functools.partial(_sc_body, b, h, w, d)
    fn = pl.kernel(
        body,
        mesh=mesh,
        out_type=jax.ShapeDtypeStruct((b * 2 * d * h * w,), jnp.float32),
        scratch_types=[
            pltpu.VMEM((2 * h * d,), jnp.float32),
            pltpu.VMEM((_LANES * h * w,), jnp.float32),
            pltpu.SemaphoreType.DMA,
        ],
    )
    out = fn(tab)
    return out.reshape(b, 2 * d, h, w)


# speedup vs baseline: 3.5792x; 3.5792x over previous
import jax
import jax.numpy as jnp
from jax.experimental import pallas as pl
from jax.experimental.pallas import tpu as pltpu

_NSRC = 4
_SPLIT = 4


def _pos_kernel(row_ref, col_ref, out_ref, p0, p1, p2, p3, sems):
    h, d = row_ref.shape
    w = col_ref.shape[0]
    hw = h * w
    b = out_ref.shape[0]
    pats = [p0, p1, p2, p3]
    lane = jax.lax.broadcasted_iota(jnp.int32, (w, hw), 1)
    sub_w = jax.lax.broadcasted_iota(jnp.int32, (w, hw), 0)
    sx = jnp.where(lane % w == sub_w, 1.0, 0.0).astype(jnp.float32)
    lane_h = jax.lax.broadcasted_iota(jnp.int32, (h, hw), 1)
    sub_h = jax.lax.broadcasted_iota(jnp.int32, (h, hw), 0)
    sy = jnp.where(lane_h // w == sub_h, 1.0, 0.0).astype(jnp.float32)
    dn = (((0,), (0,)), ((), ()))
    xm = jax.lax.dot_general(
        col_ref[...], sx, dn, preferred_element_type=jnp.float32)
    ym = jax.lax.dot_general(
        row_ref[...], sy, dn, preferred_element_type=jnp.float32)
    for p in pats:
        p[:d, :] = xm
        p[d:, :] = ym
    csz = 2 * d // _SPLIT
    copies = []
    for i in range(b):
        for j in range(_SPLIT):
            k = i * _SPLIT + j
            src = pats[k % _NSRC]
            copies.append(pltpu.make_async_copy(
                src.at[pl.ds(j * csz, csz)],
                out_ref.at[i, pl.ds(j * csz, csz)],
                sems.at[k]))
    for k, c in enumerate(copies):
        c.start(priority=k % 2)
    for c in copies:
        c.wait()


def kernel(pixel_values, row_embedding, column_embedding):
    b = pixel_values.shape[0]
    h, w = pixel_values.shape[-2], pixel_values.shape[-1]
    d = row_embedding.shape[-1]
    row = row_embedding[:h]
    col = column_embedding[:w]
    out = pl.pallas_call(
        _pos_kernel,
        in_specs=[
            pl.BlockSpec((h, d), lambda: (0, 0)),
            pl.BlockSpec((w, d), lambda: (0, 0)),
        ],
        out_specs=pl.BlockSpec(memory_space=pl.ANY),
        out_shape=jax.ShapeDtypeStruct((b, 2 * d, h * w), jnp.float32),
        scratch_shapes=[
            pltpu.VMEM((2 * d, h * w), jnp.float32)
            for _ in range(_NSRC)
        ] + [pltpu.SemaphoreType.DMA((b * _SPLIT,))],
    )(row, col)
    return out.reshape(b, 2 * d, h, w)
